# baseline (speedup 1.0000x reference)
import jax
import jax.numpy as jnp
from jax.experimental import pallas as pl
from jax.experimental.pallas import tpu as pltpu
from jax.experimental.pallas import tpu_sc as plsc

_IDX_ROWS = ((2, 7, 0), (5, 6, 3), (4, 0, 5), (1, 5, 6))


def _body(x_hbm, out_hbm, x_s, out_s):
    pltpu.sync_copy(x_hbm, x_s)
    for r, row in enumerate(_IDX_ROWS):
        for j, c in enumerate(row):
            out_s[r, j] = x_s[r, c]
    pltpu.sync_copy(out_s, out_hbm)


@jax.jit
def kernel(x):
    mesh = plsc.ScalarSubcoreMesh(axis_name="c", num_cores=1)
    return pl.kernel(
        _body,
        out_type=jax.ShapeDtypeStruct((4, 3), jnp.float32),
        mesh=mesh,
        scratch_types=[
            pltpu.SMEM((4, 8), jnp.float32),
            pltpu.SMEM((4, 3), jnp.float32),
        ],
        compiler_params=pltpu.CompilerParams(
            needs_layout_passes=False,
            skip_device_barrier=True,
            disable_bounds_checks=True,
            disable_semaphore_checks=True,
        ),
    )(x)
